# baseline (speedup 1.0000x reference)
import functools

import jax
import jax.numpy as jnp
import numpy as np
from jax import lax
from jax.experimental import pallas as pl
from jax.experimental.pallas import tpu as pltpu
from jax.experimental.pallas import tpu_sc as plsc

F, N, D, K = 4, 1024, 256, 8192
KB = 4096
NK = K // KB

NC, NS = 2, 16
NW = NC * NS
CHUNK = N // NW


def _dist_body(x_ref, w_ref, cols_ref, idx_ref, wt_ref, xsq_ref, runmin, runarg):
    kb = pl.program_id(0)

    @pl.when(kb == 0)
    def _():
        x0 = x_ref[...]
        xsq_ref[...] = jnp.sum(x0 * x0, axis=1, keepdims=True)
        runmin[...] = jnp.full((N, 128), jnp.inf, jnp.float32)
        runarg[...] = jnp.zeros((N, 128), jnp.float32)

    x = x_ref[...]
    w = w_ref[...]
    wsq = jnp.sum(w * w, axis=0, keepdims=True)
    x2 = x * -2.0
    mm2 = jnp.dot(x2, w, preferred_element_type=jnp.float32)

    xsq = xsq_ref[...]
    cols = cols_ref[...]
    m = runmin[...]
    i = runarg[...]
    kbase = jnp.float32(kb * KB)
    for j in range(KB // 128):
        sl = slice(j * 128, (j + 1) * 128)
        s = (xsq + mm2[:, sl]) + wsq[:, sl]
        lt = s < m
        m = jnp.where(lt, s, m)
        i = jnp.where(lt, cols[:, sl] + kbase, i)
    runmin[...] = m
    runarg[...] = i

    wt_ref[...] = w.T

    @pl.when(kb == NK - 1)
    def _():
        pm, pidx = runmin[...], runarg[...]
        bmin = jnp.min(pm, axis=1, keepdims=True)
        candf = jnp.where(pm == bmin, pidx, jnp.float32(K))
        barg = jnp.min(candf, axis=1, keepdims=True)
        idx_ref[...] = barg.astype(jnp.int32).reshape(N)


def _distance_argmin(x_f, w_f, cols):
    return pl.pallas_call(
        _dist_body,
        grid=(NK,),
        in_specs=[
            pl.BlockSpec((N, D), lambda kb: (0, 0)),
            pl.BlockSpec((D, KB), lambda kb: (0, kb)),
            pl.BlockSpec((1, KB), lambda kb: (0, 0)),
        ],
        out_specs=[
            pl.BlockSpec((N,), lambda kb: (0,)),
            pl.BlockSpec((KB, D), lambda kb: (kb, 0)),
        ],
        out_shape=[
            jax.ShapeDtypeStruct((N,), jnp.int32),
            jax.ShapeDtypeStruct((K, D), jnp.float32),
        ],
        scratch_shapes=[
            pltpu.VMEM((N, 1), jnp.float32),
            pltpu.VMEM((N, 128), jnp.float32),
            pltpu.VMEM((N, 128), jnp.float32),
        ],
        compiler_params=pltpu.CompilerParams(
            dimension_semantics=("arbitrary",),
        ),
    )(x_f, w_f, cols)


@functools.cache
def _make_sc_gather():
    @functools.partial(
        pl.kernel,
        mesh=plsc.VectorSubcoreMesh(core_axis_name="c", subcore_axis_name="s"),
        out_type=jax.ShapeDtypeStruct((N, D), jnp.float32),
        scratch_types=[
            pltpu.VMEM((CHUNK,), jnp.int32),
            pltpu.VMEM((CHUNK, D), jnp.float32),
            pltpu.SemaphoreType.DMA,
        ],
    )
    def _sc_gather(wt_hbm, idx_hbm, out_hbm, idx_v, rows_v, sem):
        wid = lax.axis_index("s") * NC + lax.axis_index("c")
        base = wid * CHUNK
        pltpu.sync_copy(idx_hbm.at[pl.ds(base, CHUNK)], idx_v)
        pltpu.async_copy(wt_hbm.at[idx_v], rows_v, sem).wait()
        pltpu.sync_copy(rows_v, out_hbm.at[pl.ds(base, CHUNK)])

    return _sc_gather


def kernel(inputs, w):
    cols = jnp.asarray(np.arange(KB, dtype=np.float32)[None, :])
    gather = _make_sc_gather()
    outs = []
    for f in range(F):
        gidx, wt = _distance_argmin(inputs[f], w[f], cols)
        outs.append(gather(wt, gidx))
    return jnp.stack(outs)


# speedup vs baseline: 1.6648x; 1.6648x over previous
import functools

import jax
import jax.numpy as jnp
import numpy as np
from jax import lax
from jax.experimental import pallas as pl
from jax.experimental.pallas import tpu as pltpu
from jax.experimental.pallas import tpu_sc as plsc

F, N, D, K = 4, 1024, 256, 8192
KB = 4096
NK = K // KB

NC, NS = 2, 16
NW = NC * NS
CHUNK = (F * N) // NW


def _dist_body(x_ref, w_ref, cols_ref, idx_ref, wt_ref, xsq_ref, runmin, runarg):
    kb = pl.program_id(1)

    @pl.when(kb == 0)
    def _():
        x0 = x_ref[0]
        xsq_ref[...] = jnp.sum(x0 * x0, axis=1, keepdims=True)
        runmin[...] = jnp.full((N, 128), jnp.inf, jnp.float32)
        runarg[...] = jnp.zeros((N, 128), jnp.float32)

    x = x_ref[0]
    w = w_ref[0]
    wsq = jnp.sum(w * w, axis=0, keepdims=True)
    x2 = x * -2.0
    mm2 = jnp.dot(x2, w, preferred_element_type=jnp.float32)

    xsq = xsq_ref[...]
    cols = cols_ref[...]
    m = runmin[...]
    i = runarg[...]
    kbase = jnp.float32(kb * KB)
    for j in range(KB // 128):
        sl = slice(j * 128, (j + 1) * 128)
        s = (xsq + mm2[:, sl]) + wsq[:, sl]
        lt = s < m
        m = jnp.where(lt, s, m)
        i = jnp.where(lt, cols[:, sl] + kbase, i)
    runmin[...] = m
    runarg[...] = i

    wt_ref[...] = w.T

    @pl.when(kb == NK - 1)
    def _():
        f = pl.program_id(0)
        pm, pidx = runmin[...], runarg[...]
        bmin = jnp.min(pm, axis=1, keepdims=True)
        candf = jnp.where(pm == bmin, pidx, jnp.float32(K))
        barg = jnp.min(candf, axis=1, keepdims=True)
        idx_ref[...] = (barg.astype(jnp.int32) + f * K).reshape(N)


def _distance_argmin(inputs, w):
    cols = jnp.asarray(np.arange(KB, dtype=np.float32)[None, :])
    return pl.pallas_call(
        _dist_body,
        grid=(F, NK),
        in_specs=[
            pl.BlockSpec((1, N, D), lambda f, kb: (f, 0, 0)),
            pl.BlockSpec((1, D, KB), lambda f, kb: (f, 0, kb)),
            pl.BlockSpec((1, KB), lambda f, kb: (0, 0)),
        ],
        out_specs=[
            pl.BlockSpec((N,), lambda f, kb: (f,)),
            pl.BlockSpec((KB, D), lambda f, kb: (f * NK + kb, 0)),
        ],
        out_shape=[
            jax.ShapeDtypeStruct((F * N,), jnp.int32),
            jax.ShapeDtypeStruct((F * K, D), jnp.float32),
        ],
        scratch_shapes=[
            pltpu.VMEM((N, 1), jnp.float32),
            pltpu.VMEM((N, 128), jnp.float32),
            pltpu.VMEM((N, 128), jnp.float32),
        ],
        compiler_params=pltpu.CompilerParams(
            dimension_semantics=("parallel", "arbitrary"),
        ),
    )(inputs, w, cols)


@functools.cache
def _make_sc_gather():
    @functools.partial(
        pl.kernel,
        mesh=plsc.VectorSubcoreMesh(core_axis_name="c", subcore_axis_name="s"),
        out_type=jax.ShapeDtypeStruct((F * N, D), jnp.float32),
        scratch_types=[
            pltpu.VMEM((CHUNK,), jnp.int32),
            pltpu.VMEM((CHUNK, D), jnp.float32),
            pltpu.SemaphoreType.DMA,
        ],
    )
    def _sc_gather(wt_hbm, idx_hbm, out_hbm, idx_v, rows_v, sem):
        wid = lax.axis_index("s") * NC + lax.axis_index("c")
        base = wid * CHUNK
        pltpu.sync_copy(idx_hbm.at[pl.ds(base, CHUNK)], idx_v)
        pltpu.async_copy(wt_hbm.at[idx_v], rows_v, sem).wait()
        pltpu.sync_copy(rows_v, out_hbm.at[pl.ds(base, CHUNK)])

    return _sc_gather


def kernel(inputs, w):
    gidx, wt = _distance_argmin(inputs, w)
    out = _make_sc_gather()(wt, gidx)
    return out.reshape(F, N, D)
